# baseline (speedup 1.0000x reference)
import functools

import jax
import jax.numpy as jnp
from jax.experimental import pallas as pl


C = 32
NB = 8


def _pp_body(p1_ref, wpp_ref, out_ref):
    out_ref[...] = jnp.tanh(
        jnp.dot(p1_ref[...], wpp_ref[...], preferred_element_type=jnp.float32)
    )


def _mlp_body(xi_ref, xj_ref, basis_ref, wpi_ref, wii_ref, out_ref):
    x = (xi_ref[...] + xj_ref[...]).astype(jnp.bfloat16)
    h = jnp.tanh(jnp.dot(x, wpi_ref[...], preferred_element_type=jnp.float32))
    acc = h[:, 0:C] * basis_ref[:, 0:1]
    for b in range(1, NB):
        acc = acc + h[:, b * C:(b + 1) * C] * basis_ref[:, b:b + 1]
    i1 = jnp.tanh(
        jnp.dot(acc.astype(jnp.bfloat16), wii_ref[...],
                preferred_element_type=jnp.float32)
    )
    out_ref[...] = i1


def _pp_call(p1f, wpp, nb):
    n = p1f.shape[0]
    return pl.pallas_call(
        _pp_body,
        grid=(n // nb,),
        in_specs=[
            pl.BlockSpec((nb, C), lambda i: (i, 0)),
            pl.BlockSpec((C, C), lambda i: (0, 0)),
        ],
        out_specs=pl.BlockSpec((nb, C), lambda i: (i, 0)),
        out_shape=jax.ShapeDtypeStruct((n, C), jnp.float32),
    )(p1f, wpp)


def _mlp_call(xi, xj, basis, wpi2, wii, eb):
    e = xi.shape[0]
    return pl.pallas_call(
        _mlp_body,
        grid=(e // eb,),
        in_specs=[
            pl.BlockSpec((eb, C), lambda i: (i, 0)),
            pl.BlockSpec((eb, C), lambda i: (i, 0)),
            pl.BlockSpec((eb, NB), lambda i: (i, 0)),
            pl.BlockSpec((C, C * NB), lambda i: (0, 0)),
            pl.BlockSpec((C, C), lambda i: (0, 0)),
        ],
        out_specs=pl.BlockSpec((eb, C), lambda i: (i, 0)),
        out_shape=jax.ShapeDtypeStruct((e, C), jnp.float32),
    )(xi, xj, basis, wpi2, wii)


def kernel(pair_i, pair_j, p1, basis, W_pp, W_pi, W_ii):
    n = p1.shape[0]
    e = basis.shape[0]
    p1f = p1.reshape(n, C)

    p1p = _pp_call(p1f, W_pp, nb=10000)

    xi = jnp.take(p1p, pair_i, axis=0)
    xj = jnp.take(p1p, pair_j, axis=0)

    wpi2 = (W_pi.reshape(C, C, NB).transpose(0, 2, 1).reshape(C, C * NB)
            .astype(jnp.bfloat16))
    i1 = _mlp_call(xi, xj, basis, wpi2, W_ii.astype(jnp.bfloat16), eb=4000)

    p1_new = jax.ops.segment_sum(i1, pair_i, num_segments=n)
    return (p1_new.reshape(n, 1, C), i1.reshape(e, 1, C))


# speedup vs baseline: 2.4234x; 2.4234x over previous
import functools

import jax
import jax.numpy as jnp
from jax import lax
from jax.experimental import pallas as pl
from jax.experimental.pallas import tpu as pltpu
from jax.experimental.pallas import tpu_sc as plsc


C = 32
NB = 8

_NC = 2
_NS = 16
_NW = _NC * _NS

_GCHUNK = 2000
_SCHUNK = 800

_MESH = plsc.VectorSubcoreMesh(core_axis_name="c", subcore_axis_name="s")
_SC_PARAMS = pltpu.CompilerParams(use_tc_tiling_on_sc=False)


def _pp_body(p1_ref, wpp_ref, out_ref):
    out_ref[...] = jnp.tanh(
        jnp.dot(p1_ref[...], wpp_ref[...], preferred_element_type=jnp.float32)
    )


def _pp_call(p1f, wpp, nb):
    n = p1f.shape[0]
    return pl.pallas_call(
        _pp_body,
        grid=(n // nb,),
        in_specs=[
            pl.BlockSpec((nb, C), lambda i: (i, 0)),
            pl.BlockSpec((C, C), lambda i: (0, 0)),
        ],
        out_specs=pl.BlockSpec((nb, C), lambda i: (i, 0)),
        out_shape=jax.ShapeDtypeStruct((n, C), jnp.float32),
    )(p1f, wpp)


def _gather_call(tab, pair_i, pair_j):
    e = pair_i.shape[0]
    per_w = e // _NW

    @functools.partial(
        pl.kernel,
        out_type=(
            jax.ShapeDtypeStruct((e, C), jnp.float32),
            jax.ShapeDtypeStruct((e, C), jnp.float32),
        ),
        mesh=_MESH,
        scratch_types=[
            pltpu.VMEM((_GCHUNK,), jnp.int32),
            pltpu.VMEM((_GCHUNK, C), jnp.float32),
            pltpu.SemaphoreType.DMA,
        ],
        compiler_params=_SC_PARAMS,
    )
    def k(tab_hbm, pi_hbm, pj_hbm, xi_hbm, xj_hbm, idx_v, rows_v, sem):
        wid = lax.axis_index("s") * _NC + lax.axis_index("c")
        base = wid * per_w

        @pl.loop(0, per_w, step=_GCHUNK)
        def _(off):
            start = base + off
            pltpu.sync_copy(pi_hbm.at[pl.ds(start, _GCHUNK)], idx_v)
            pltpu.async_copy(tab_hbm.at[idx_v], rows_v, sem).wait()
            pltpu.sync_copy(rows_v, xi_hbm.at[pl.ds(start, _GCHUNK)])
            pltpu.sync_copy(pj_hbm.at[pl.ds(start, _GCHUNK)], idx_v)
            pltpu.async_copy(tab_hbm.at[idx_v], rows_v, sem).wait()
            pltpu.sync_copy(rows_v, xj_hbm.at[pl.ds(start, _GCHUNK)])

    return k(tab, pair_i, pair_j)


def _mlp_body(xi_ref, xj_ref, basis_ref, wpi_ref, wii_ref, out_ref):
    x = (xi_ref[...] + xj_ref[...]).astype(jnp.bfloat16)
    h = jnp.tanh(jnp.dot(x, wpi_ref[...], preferred_element_type=jnp.float32))
    acc = h[:, 0:C] * basis_ref[:, 0:1]
    for b in range(1, NB):
        acc = acc + h[:, b * C:(b + 1) * C] * basis_ref[:, b:b + 1]
    i1 = jnp.tanh(
        jnp.dot(acc.astype(jnp.bfloat16), wii_ref[...],
                preferred_element_type=jnp.float32)
    )
    out_ref[...] = i1


def _mlp_call(xi, xj, basis, wpi2, wii, eb):
    e = xi.shape[0]
    return pl.pallas_call(
        _mlp_body,
        grid=(e // eb,),
        in_specs=[
            pl.BlockSpec((eb, C), lambda i: (i, 0)),
            pl.BlockSpec((eb, C), lambda i: (i, 0)),
            pl.BlockSpec((eb, NB), lambda i: (i, 0)),
            pl.BlockSpec((C, C * NB), lambda i: (0, 0)),
            pl.BlockSpec((C, C), lambda i: (0, 0)),
        ],
        out_specs=pl.BlockSpec((eb, C), lambda i: (i, 0)),
        out_shape=jax.ShapeDtypeStruct((e, C), jnp.float32),
    )(xi, xj, basis, wpi2, wii)


def _scatter_call(i1, pair_i, n):
    e = pair_i.shape[0]
    half = n // _NC
    acc_rows = half + 48
    trash = half + 40
    stripe = 3128
    per_s = e // _NS
    zeros = jnp.zeros((acc_rows, C), jnp.float32)

    @functools.partial(
        pl.kernel,
        out_type=jax.ShapeDtypeStruct((n, C), jnp.float32),
        mesh=_MESH,
        scratch_types=[
            pltpu.VMEM((_SCHUNK,), jnp.int32),
            pltpu.VMEM((_SCHUNK, C), jnp.float32),
            pltpu.VMEM_SHARED((acc_rows, C), jnp.float32),
        ],
        compiler_params=_SC_PARAMS,
    )
    def k(i1_hbm, pi_hbm, z_hbm, out_hbm, idx_v, rows_v, acc_sh):
        cid = lax.axis_index("c")
        sid = lax.axis_index("s")
        node_base = cid * half

        zstart = sid * stripe
        pltpu.sync_copy(z_hbm.at[pl.ds(zstart, stripe)],
                        acc_sh.at[pl.ds(zstart, stripe)])
        plsc.subcore_barrier()

        ebase = sid * per_s

        @pl.loop(0, per_s, step=_SCHUNK)
        def _(off):
            start = ebase + off
            pltpu.sync_copy(pi_hbm.at[pl.ds(start, _SCHUNK)], idx_v)
            pltpu.sync_copy(i1_hbm.at[pl.ds(start, _SCHUNK)], rows_v)

            @pl.loop(0, _SCHUNK, step=16)
            def _(t):
                v = idx_v[pl.ds(t, 16)]
                ok = (v >= node_base) & (v < node_base + half)
                idx_v[pl.ds(t, 16)] = jnp.where(ok, v - node_base, trash)

            pltpu.sync_copy(rows_v, acc_sh.at[idx_v], add=True)

        plsc.subcore_barrier()

        wlen_last = half - 15 * stripe
        obase = node_base + sid * stripe

        @pl.when(sid < _NS - 1)
        def _():
            pltpu.sync_copy(acc_sh.at[pl.ds(sid * stripe, stripe)],
                            out_hbm.at[pl.ds(obase, stripe)])

        @pl.when(sid == _NS - 1)
        def _():
            pltpu.sync_copy(acc_sh.at[pl.ds(sid * stripe, wlen_last)],
                            out_hbm.at[pl.ds(obase, wlen_last)])

    return k(i1, pair_i, zeros)


def kernel(pair_i, pair_j, p1, basis, W_pp, W_pi, W_ii):
    n = p1.shape[0]
    e = basis.shape[0]
    p1f = p1.reshape(n, C)

    p1p = _pp_call(p1f, W_pp, nb=10000)

    xi, xj = _gather_call(p1p, pair_i, pair_j)

    wpi2 = (W_pi.reshape(C, C, NB).transpose(0, 2, 1).reshape(C, C * NB)
            .astype(jnp.bfloat16))
    i1 = _mlp_call(xi, xj, basis, wpi2, W_ii.astype(jnp.bfloat16), eb=4000)

    p1_new = _scatter_call(i1, pair_i, n)
    return (p1_new.reshape(n, 1, C), i1.reshape(e, 1, C))


# speedup vs baseline: 6.3516x; 2.6210x over previous
import functools

import jax
import jax.numpy as jnp
from jax import lax
from jax.experimental import pallas as pl
from jax.experimental.pallas import tpu as pltpu
from jax.experimental.pallas import tpu_sc as plsc


C = 32
NB = 8

_NC = 2
_NS = 16
_NW = _NC * _NS

_GCHUNK = 2000
_SCHUNK = 800

_MESH = plsc.VectorSubcoreMesh(core_axis_name="c", subcore_axis_name="s")
_SC_PARAMS = pltpu.CompilerParams(use_tc_tiling_on_sc=False)


def _pp_body(p1_ref, wpp4_ref, out_ref):
    out_ref[...] = jnp.tanh(
        jnp.dot(p1_ref[...], wpp4_ref[...], preferred_element_type=jnp.float32)
    )


def _pp_call(p14, wpp4, nb4):
    n4 = p14.shape[0]
    return pl.pallas_call(
        _pp_body,
        grid=(n4 // nb4,),
        in_specs=[
            pl.BlockSpec((nb4, 4 * C), lambda i: (i, 0)),
            pl.BlockSpec((4 * C, 4 * C), lambda i: (0, 0)),
        ],
        out_specs=pl.BlockSpec((nb4, 4 * C), lambda i: (i, 0)),
        out_shape=jax.ShapeDtypeStruct((n4, 4 * C), jnp.float32),
    )(p14, wpp4)


def _gather_call(tab, pair_i, pair_j):
    e = pair_i.shape[0]
    per_w = e // _NW

    @functools.partial(
        pl.kernel,
        out_type=(
            jax.ShapeDtypeStruct((e, C), jnp.float32),
            jax.ShapeDtypeStruct((e, C), jnp.float32),
        ),
        mesh=_MESH,
        scratch_types=[
            pltpu.VMEM((_GCHUNK,), jnp.int32),
            pltpu.VMEM((_GCHUNK, C), jnp.float32),
            pltpu.SemaphoreType.DMA,
        ],
        compiler_params=_SC_PARAMS,
    )
    def k(tab_hbm, pi_hbm, pj_hbm, xi_hbm, xj_hbm, idx_v, rows_v, sem):
        wid = lax.axis_index("s") * _NC + lax.axis_index("c")
        base = wid * per_w

        @pl.loop(0, per_w, step=_GCHUNK)
        def _(off):
            start = base + off
            pltpu.sync_copy(pi_hbm.at[pl.ds(start, _GCHUNK)], idx_v)
            pltpu.async_copy(tab_hbm.at[idx_v], rows_v, sem).wait()
            pltpu.sync_copy(rows_v, xi_hbm.at[pl.ds(start, _GCHUNK)])
            pltpu.sync_copy(pj_hbm.at[pl.ds(start, _GCHUNK)], idx_v)
            pltpu.async_copy(tab_hbm.at[idx_v], rows_v, sem).wait()
            pltpu.sync_copy(rows_v, xj_hbm.at[pl.ds(start, _GCHUNK)])

    return k(tab, pair_i, pair_j)


def _mlp_body(xi_ref, xj_ref, basis_ref, wpi4_ref, r_ref, t_ref, out_ref):
    x4 = (xi_ref[...] + xj_ref[...]).astype(jnp.bfloat16)
    h4 = jnp.tanh(
        jnp.dot(x4, wpi4_ref[...], preferred_element_type=jnp.float32))
    brep = jnp.dot(basis_ref[...].astype(jnp.bfloat16), r_ref[...],
                   preferred_element_type=jnp.float32)
    prod = (h4 * brep).astype(jnp.bfloat16)
    i1 = jnp.tanh(
        jnp.dot(prod, t_ref[...], preferred_element_type=jnp.float32))
    out_ref[...] = i1


def _mlp_call(xi4, xj4, basis4, wpi4, r4, t4, eb4):
    e4 = xi4.shape[0]
    return pl.pallas_call(
        _mlp_body,
        grid=(e4 // eb4,),
        in_specs=[
            pl.BlockSpec((eb4, 4 * C), lambda i: (i, 0)),
            pl.BlockSpec((eb4, 4 * C), lambda i: (i, 0)),
            pl.BlockSpec((eb4, 4 * NB), lambda i: (i, 0)),
            pl.BlockSpec((4 * C, 4 * C * NB), lambda i: (0, 0)),
            pl.BlockSpec((4 * NB, 4 * C * NB), lambda i: (0, 0)),
            pl.BlockSpec((4 * C * NB, 4 * C), lambda i: (0, 0)),
        ],
        out_specs=pl.BlockSpec((eb4, 4 * C), lambda i: (i, 0)),
        out_shape=jax.ShapeDtypeStruct((e4, 4 * C), jnp.float32),
    )(xi4, xj4, basis4, wpi4, r4, t4)


def _scatter_call(i1, pair_i, n):
    e = pair_i.shape[0]
    half = n // _NC
    acc_rows = half + 48
    stripe = acc_rows // _NS
    per_s = e // _NS
    zeros = jnp.zeros((acc_rows, C), jnp.float32)

    @functools.partial(
        pl.kernel,
        out_type=jax.ShapeDtypeStruct((n, C), jnp.float32),
        mesh=_MESH,
        scratch_types=[
            pltpu.VMEM((_SCHUNK,), jnp.int32),
            pltpu.VMEM((_SCHUNK, C), jnp.float32),
            pltpu.VMEM_SHARED((acc_rows, C), jnp.float32),
        ],
        compiler_params=_SC_PARAMS,
    )
    def k(i1_hbm, pi_hbm, z_hbm, out_hbm, idx_v, rows_v, acc_sh):
        cid = lax.axis_index("c")
        sid = lax.axis_index("s")
        node_base = cid * half

        zstart = sid * stripe
        pltpu.sync_copy(z_hbm.at[pl.ds(zstart, stripe)],
                        acc_sh.at[pl.ds(zstart, stripe)])
        plsc.subcore_barrier()

        ebase = sid * per_s
        trash_v = half + lax.iota(jnp.int32, 16)

        @pl.loop(0, per_s, step=_SCHUNK)
        def _(off):
            start = ebase + off
            pltpu.sync_copy(pi_hbm.at[pl.ds(start, _SCHUNK)], idx_v)
            pltpu.sync_copy(i1_hbm.at[pl.ds(start, _SCHUNK)], rows_v)

            @pl.loop(0, _SCHUNK, step=16)
            def _(t):
                v = idx_v[pl.ds(t, 16)]
                ok = (v >= node_base) & (v < node_base + half)
                idx_v[pl.ds(t, 16)] = jnp.where(ok, v - node_base, trash_v)

            pltpu.sync_copy(rows_v, acc_sh.at[idx_v], add=True)

        plsc.subcore_barrier()

        wlen_last = half - (_NS - 1) * stripe
        obase = node_base + sid * stripe

        @pl.when(sid < _NS - 1)
        def _():
            pltpu.sync_copy(acc_sh.at[pl.ds(sid * stripe, stripe)],
                            out_hbm.at[pl.ds(obase, stripe)])

        @pl.when(sid == _NS - 1)
        def _():
            pltpu.sync_copy(acc_sh.at[pl.ds(sid * stripe, wlen_last)],
                            out_hbm.at[pl.ds(obase, wlen_last)])

    return k(i1, pair_i, zeros)


def _pack_weights(W_pp, W_pi, W_ii):
    eye4 = jnp.eye(4, dtype=jnp.float32)
    wpp4 = (jnp.einsum("kK,cd->kcKd", eye4, W_pp).reshape(4 * C, 4 * C))
    wpi2 = W_pi.reshape(C, C, NB).transpose(0, 2, 1).reshape(C, C * NB)
    wpi4 = (jnp.einsum("kK,cj->kcKj", eye4, wpi2)
            .reshape(4 * C, 4 * C * NB).astype(jnp.bfloat16))
    eye8 = jnp.eye(NB, dtype=jnp.float32)
    ones_c = jnp.ones((C,), jnp.float32)
    r4 = (jnp.einsum("kK,bB,c->kbKBc", eye4, eye8, ones_c)
          .reshape(4 * NB, 4 * C * NB).astype(jnp.bfloat16))
    t4 = (jnp.broadcast_to(
        (eye4[:, None, None, :, None] * W_ii[None, None, :, None, :]),
        (4, NB, C, 4, C)).reshape(4 * C * NB, 4 * C).astype(jnp.bfloat16))
    return wpp4, wpi4, r4, t4


def kernel(pair_i, pair_j, p1, basis, W_pp, W_pi, W_ii):
    n = p1.shape[0]
    e = basis.shape[0]

    wpp4, wpi4, r4, t4 = _pack_weights(W_pp, W_pi, W_ii)

    p14 = p1.reshape(n // 4, 4 * C)
    tab4 = _pp_call(p14, wpp4, nb4=5000)

    xi, xj = _gather_call(tab4.reshape(n, C), pair_i, pair_j)

    i1_4 = _mlp_call(
        xi.reshape(e // 4, 4 * C),
        xj.reshape(e // 4, 4 * C),
        basis.reshape(e // 4, 4 * NB),
        wpi4, r4, t4, eb4=1000)

    p1_new = _scatter_call(i1_4.reshape(e, C), pair_i, n)
    return (p1_new.reshape(n, 1, C), i1_4.reshape(e, 1, C))
